# baseline (speedup 1.0000x reference)
import functools
import math
import jax
import jax.numpy as jnp
from jax.experimental import pallas as pl
from jax.experimental.pallas import tpu as pltpu

C1 = 150; D = 32; NH = 4; KCH = 10; M = 12; H = 14; W = 14; RATIO = 0.08
N = H * W
KGLOBAL = max(1, int(math.ceil(RATIO * (C1 * H * W))))
BB = 8
MP = 16


def _pos2d(h, w, dim):
    d2 = dim // 2
    div = jnp.exp(jnp.arange(0, d2, 2, dtype=jnp.float32) * -(math.log(10000.0) / d2))
    pos_y = jnp.arange(h, dtype=jnp.float32)[:, None]
    ang_y = pos_y * div[None, :]
    pe_y = jnp.zeros((h, d2), jnp.float32).at[:, 0::2].set(jnp.sin(ang_y)).at[:, 1::2].set(jnp.cos(ang_y))
    pos_x = jnp.arange(w, dtype=jnp.float32)[:, None]
    ang_x = pos_x * div[None, :]
    pe_x = jnp.zeros((w, d2), jnp.float32).at[:, 0::2].set(jnp.sin(ang_x)).at[:, 1::2].set(jnp.cos(ang_x))
    pe = jnp.concatenate([
        jnp.broadcast_to(pe_y[:, None, :], (h, w, d2)),
        jnp.broadcast_to(pe_x[None, :, :], (h, w, d2)),
    ], axis=-1)
    return pe.reshape(h * w, dim)


def _dot(a, b, dims):
    return jax.lax.dot_general(a, b, (dims, ((), ())),
                               preferred_element_type=jnp.float32)


def _fused_kernel(pt_ref, w2_ref, cemb_ref, pe_ref, inw_ref, inb_ref,
                  outw_ref, outb_ref, slots_ref, projw_ref,
                  z_ref, am_ref, het_ref, sw_ref, ti_ref, ss_ref, tmt_ref):
    bb = pt_ref.shape[0]
    w2 = w2_ref[...]

    a_list = []
    for j in range(bb):
        aj = _dot(pt_ref[j], w2, (((0,), (0,))))
        a_list.append(jnp.maximum(aj, 0.0))
    A = jnp.stack(a_list, axis=0)

    abits = jnp.maximum(jax.lax.bitcast_convert_type(A, jnp.int32), 0)
    gmaxb = jnp.max(abits, axis=(1, 2), keepdims=True)

    cif = jax.lax.broadcasted_iota(jnp.int32, (bb, N, C1), 2).astype(jnp.float32)
    m = A
    cati, catv = [], []
    for _ in range(KCH):
        cur = jnp.max(m, axis=2, keepdims=True)
        idxf = jnp.min(jnp.where(m == cur, cif, jnp.float32(C1)),
                       axis=2, keepdims=True)
        cati.append(idxf)
        catv.append(cur)
        m = jnp.where(cif == idxf, jnp.float32(-1.0), m)

    lo = jnp.zeros((bb, 1, 1), jnp.int32)
    hi = gmaxb + 1
    for _ in range(31):
        mid = lo + jax.lax.shift_right_logical(hi - lo, 1)
        cnt = jnp.sum(jnp.where(abits >= mid, 1.0, 0.0),
                      axis=(1, 2), keepdims=True)
        ge = cnt >= float(KGLOBAL)
        lo = jnp.where(ge, mid, lo)
        hi = jnp.where(ge, hi, mid)
    thresh = jax.lax.bitcast_convert_type(lo, jnp.float32)

    WIN = 24
    alive = [catv[r] >= thresh for r in range(KCH)]
    s = jnp.zeros((bb, N, 1), jnp.float32)
    for r in range(KCH):
        s = s + jnp.where(alive[r], 1.0, 0.0)
    cwin = jax.lax.broadcasted_iota(jnp.int32, (bb, N, WIN), 2).astype(jnp.float32)
    inS = jnp.zeros((bb, N, WIN), jnp.float32)
    for r in range(KCH):
        inS = inS + jnp.where((cwin == cati[r]) & alive[r], 1.0, 0.0)
    notS = 1.0 - inS
    cum = notS
    for sh in (1, 2, 4, 8, 16):
        cum = cum + jnp.concatenate(
            [jnp.zeros((bb, N, sh), jnp.float32), cum[..., :WIN - sh]], axis=2)
    rankex = cum - notS
    tis = []
    for r in range(KCH):
        p = jnp.float32(r) - s
        fr = jnp.min(jnp.where((notS > 0.0) & (rankex == p), cwin,
                               jnp.float32(C1)), axis=2, keepdims=True)
        tis.append(jnp.where(alive[r], cati[r], fr).astype(jnp.int32))
    ti_ref[...] = jnp.concatenate(tis, axis=2)

    gmaxf = jax.lax.bitcast_convert_type(gmaxb, jnp.float32)
    denom = jnp.where(gmaxf == 0.0, 1.0, gmaxf)
    sw = jnp.where((m < 0.0) & (A >= thresh), A, 0.0) / denom
    sw_ref[...] = sw

    cemb = cemb_ref[...]; pe = pe_ref[...]
    inw = inw_ref[...]; inb = inb_ref[...]
    outw = outw_ref[...]; outb = outb_ref[...]
    slots = slots_ref[...]
    dh = D // NH
    asc = 1.0 / math.sqrt(dh)
    ssc = 1.0 / math.sqrt(D)
    z_rows, he_cols, tm_cols = [], [], []
    for j in range(bb):
        tok = _dot(sw[j], cemb, (((1,), (0,)))) + pe
        qkv = _dot(tok, inw, (((1,), (1,)))) + inb
        heads = []
        for h in range(NH):
            q = qkv[:, dh * h:dh * h + dh]
            k = qkv[:, D + dh * h:D + dh * h + dh]
            v = qkv[:, 2 * D + dh * h:2 * D + dh * h + dh]
            sc = _dot(q, k, (((1,), (1,)))) * asc
            sc = sc - jnp.max(sc, axis=1, keepdims=True)
            e = jnp.exp(sc)
            p = e / jnp.sum(e, axis=1, keepdims=True)
            heads.append(_dot(p, v, (((1,), (0,)))))
        o = jnp.concatenate(heads, axis=1)
        ao = _dot(o, outw, (((1,), (1,)))) + outb
        st = _dot(slots, ao, (((1,), (1,)))) * ssc
        st = st - jnp.max(st, axis=1, keepdims=True)
        e = jnp.exp(st)
        am16 = e / jnp.sum(e, axis=1, keepdims=True)
        am = am16[:M]
        am_ref[j] = am
        he_cols.append(jnp.sum(am * am, axis=1, keepdims=True))
        ssj = _dot(am, ao, (((1,), (0,))))
        ss_ref[j] = ssj
        ni = jax.lax.broadcasted_iota(jnp.int32, (M, N), 1).astype(jnp.float32)
        mm = am
        acc = jnp.zeros((M, 1), jnp.float32)
        for _ in range(16):
            cur = jnp.max(mm, axis=1, keepdims=True)
            ii = jnp.min(jnp.where(mm == cur, ni, jnp.float32(N)),
                         axis=1, keepdims=True)
            acc = acc + cur
            mm = jnp.where(ni == ii, jnp.float32(-1.0), mm)
        tm_cols.append(acc)
        z_rows.append(jnp.mean(ssj, axis=0, keepdims=True))
    het_ref[0] = jnp.concatenate(he_cols, axis=1)
    tmt_ref[0] = jnp.concatenate(tm_cols, axis=1)
    zin = jnp.concatenate(z_rows, axis=0)
    z_ref[...] = _dot(zin, projw_ref[...], (((1,), (0,))))


def kernel(x, conv_w, channel_embed, in_proj_w, in_proj_b, out_proj_w,
           out_proj_b, slots, proj_w):
    Bn = x.shape[0]
    patches = jax.lax.conv_general_dilated_patches(
        x, (9, 9), (2, 2), ((4, 4), (4, 4)),
        dimension_numbers=('NCHW', 'OIHW', 'NCHW'))
    pt = patches.reshape(Bn, 81, N)
    w2 = conv_w.reshape(C1, 81).T
    pe = _pos2d(H, W, D)

    grid = (Bn // BB,)
    outs = pl.pallas_call(
        _fused_kernel,
        grid=grid,
        in_specs=[
            pl.BlockSpec((BB, 81, N), lambda i: (i, 0, 0)),
            pl.BlockSpec((81, C1), lambda i: (0, 0)),
            pl.BlockSpec((C1, D), lambda i: (0, 0)),
            pl.BlockSpec((N, D), lambda i: (0, 0)),
            pl.BlockSpec((3 * D, D), lambda i: (0, 0)),
            pl.BlockSpec((1, 3 * D), lambda i: (0, 0)),
            pl.BlockSpec((D, D), lambda i: (0, 0)),
            pl.BlockSpec((1, D), lambda i: (0, 0)),
            pl.BlockSpec((MP, D), lambda i: (0, 0)),
            pl.BlockSpec((D, D), lambda i: (0, 0)),
        ],
        out_specs=[
            pl.BlockSpec((BB, D), lambda i: (i, 0)),
            pl.BlockSpec((BB, M, N), lambda i: (i, 0, 0)),
            pl.BlockSpec((1, M, BB), lambda i: (i, 0, 0)),
            pl.BlockSpec((BB, N, C1), lambda i: (i, 0, 0)),
            pl.BlockSpec((BB, N, KCH), lambda i: (i, 0, 0)),
            pl.BlockSpec((BB, M, D), lambda i: (i, 0, 0)),
            pl.BlockSpec((1, M, BB), lambda i: (i, 0, 0)),
        ],
        out_shape=[
            jax.ShapeDtypeStruct((Bn, D), jnp.float32),
            jax.ShapeDtypeStruct((Bn, M, N), jnp.float32),
            jax.ShapeDtypeStruct((Bn // BB, M, BB), jnp.float32),
            jax.ShapeDtypeStruct((Bn, N, C1), jnp.float32),
            jax.ShapeDtypeStruct((Bn, N, KCH), jnp.int32),
            jax.ShapeDtypeStruct((Bn, M, D), jnp.float32),
            jax.ShapeDtypeStruct((Bn // BB, M, BB), jnp.float32),
        ],
        compiler_params=pltpu.CompilerParams(
            dimension_semantics=("arbitrary",)),
    )(pt, w2, channel_embed, pe, in_proj_w, in_proj_b.reshape(1, 3 * D),
      out_proj_w, out_proj_b.reshape(1, D),
      jnp.concatenate([slots, jnp.zeros((MP - M, D), jnp.float32)], axis=0),
      proj_w)
    z, am, het, sw, ti, ss, tmt = outs
    sparse_weights = sw.reshape(Bn, H, W, C1)
    topi = ti.reshape(Bn, H, W, KCH)
    he = het.transpose(0, 2, 1).reshape(Bn, M)
    tm = tmt.transpose(0, 2, 1).reshape(Bn, M)
    return (z, am, he, sparse_weights, topi, ss, tm)


# speedup vs baseline: 1.0342x; 1.0342x over previous
import functools
import math
import jax
import jax.numpy as jnp
from jax.experimental import pallas as pl
from jax.experimental.pallas import tpu as pltpu

C1 = 150; D = 32; NH = 4; KCH = 10; M = 12; H = 14; W = 14; RATIO = 0.08
N = H * W
KGLOBAL = max(1, int(math.ceil(RATIO * (C1 * H * W))))
BB = 8
MP = 16


def _pos2d(h, w, dim):
    d2 = dim // 2
    div = jnp.exp(jnp.arange(0, d2, 2, dtype=jnp.float32) * -(math.log(10000.0) / d2))
    pos_y = jnp.arange(h, dtype=jnp.float32)[:, None]
    ang_y = pos_y * div[None, :]
    pe_y = jnp.zeros((h, d2), jnp.float32).at[:, 0::2].set(jnp.sin(ang_y)).at[:, 1::2].set(jnp.cos(ang_y))
    pos_x = jnp.arange(w, dtype=jnp.float32)[:, None]
    ang_x = pos_x * div[None, :]
    pe_x = jnp.zeros((w, d2), jnp.float32).at[:, 0::2].set(jnp.sin(ang_x)).at[:, 1::2].set(jnp.cos(ang_x))
    pe = jnp.concatenate([
        jnp.broadcast_to(pe_y[:, None, :], (h, w, d2)),
        jnp.broadcast_to(pe_x[None, :, :], (h, w, d2)),
    ], axis=-1)
    return pe.reshape(h * w, dim)


def _dot(a, b, dims):
    return jax.lax.dot_general(a, b, (dims, ((), ())),
                               preferred_element_type=jnp.float32)


def _fused_kernel(pt_ref, w2_ref, cemb_ref, pe_ref, inw_ref, inb_ref,
                  outw_ref, outb_ref, slots_ref, projw_ref,
                  z_ref, am_ref, het_ref, sw_ref, ti_ref, ss_ref, tmt_ref):
    bb = pt_ref.shape[0]
    w2 = w2_ref[...]

    a_list = []
    for j in range(bb):
        aj = _dot(pt_ref[j], w2, (((0,), (0,))))
        a_list.append(jnp.maximum(aj, 0.0))
    A = jnp.stack(a_list, axis=0)

    abits = jnp.maximum(jax.lax.bitcast_convert_type(A, jnp.int32), 0)
    gmaxb = jnp.max(abits, axis=(1, 2), keepdims=True)

    lo = jnp.zeros((bb, 1, 1), jnp.int32)
    hi = gmaxb + 1
    for _ in range(31):
        mid = lo + jax.lax.shift_right_logical(hi - lo, 1)
        cnt = jnp.sum(jnp.where(abits >= mid, 1.0, 0.0),
                      axis=(1, 2), keepdims=True)
        ge = cnt >= float(KGLOBAL)
        lo = jnp.where(ge, mid, lo)
        hi = jnp.where(ge, hi, mid)
    thresh = jax.lax.bitcast_convert_type(lo, jnp.float32)
    m0 = jnp.where(A >= thresh, A, 0.0)

    cif = jax.lax.broadcasted_iota(jnp.int32, (bb, N, C1), 2).astype(jnp.float32)
    m = m0
    idxs = []
    for _ in range(KCH):
        cur = jnp.max(m, axis=2, keepdims=True)
        idxf = jnp.min(jnp.where(m == cur, cif, jnp.float32(C1)),
                       axis=2, keepdims=True)
        idxs.append(idxf.astype(jnp.int32))
        m = jnp.where(cif == idxf, jnp.float32(-1.0), m)
    ti_ref[...] = jnp.concatenate(idxs, axis=2)

    gmaxf = jax.lax.bitcast_convert_type(gmaxb, jnp.float32)
    denom = jnp.where(gmaxf == 0.0, 1.0, gmaxf)
    sw = jnp.where(m < 0.0, m0, 0.0) / denom
    sw_ref[...] = sw

    cemb = cemb_ref[...]; pe = pe_ref[...]
    inw = inw_ref[...]; inb = inb_ref[...]
    outw = outw_ref[...]; outb = outb_ref[...]
    slots = slots_ref[...]
    dh = D // NH
    asc = 1.0 / math.sqrt(dh)
    ssc = 1.0 / math.sqrt(D)
    z_rows, he_cols, tm_cols = [], [], []
    for j in range(bb):
        tok = _dot(sw[j], cemb, (((1,), (0,)))) + pe
        qkv = _dot(tok, inw, (((1,), (1,)))) + inb
        heads = []
        for h in range(NH):
            q = qkv[:, dh * h:dh * h + dh]
            k = qkv[:, D + dh * h:D + dh * h + dh]
            v = qkv[:, 2 * D + dh * h:2 * D + dh * h + dh]
            sc = _dot(q, k, (((1,), (1,)))) * asc
            sc = sc - jnp.max(sc, axis=1, keepdims=True)
            e = jnp.exp(sc)
            p = e / jnp.sum(e, axis=1, keepdims=True)
            heads.append(_dot(p, v, (((1,), (0,)))))
        o = jnp.concatenate(heads, axis=1)
        ao = _dot(o, outw, (((1,), (1,)))) + outb
        st = _dot(slots, ao, (((1,), (1,)))) * ssc
        st = st - jnp.max(st, axis=1, keepdims=True)
        e = jnp.exp(st)
        am16 = e / jnp.sum(e, axis=1, keepdims=True)
        am = am16[:M]
        am_ref[j] = am
        he_cols.append(jnp.sum(am * am, axis=1, keepdims=True))
        ssj = _dot(am, ao, (((1,), (0,))))
        ss_ref[j] = ssj
        ni = jax.lax.broadcasted_iota(jnp.int32, (M, N), 1).astype(jnp.float32)
        mm = am
        acc = jnp.zeros((M, 1), jnp.float32)
        for _ in range(16):
            cur = jnp.max(mm, axis=1, keepdims=True)
            ii = jnp.min(jnp.where(mm == cur, ni, jnp.float32(N)),
                         axis=1, keepdims=True)
            acc = acc + cur
            mm = jnp.where(ni == ii, jnp.float32(-1.0), mm)
        tm_cols.append(acc)
        z_rows.append(jnp.mean(ssj, axis=0, keepdims=True))
    het_ref[0] = jnp.concatenate(he_cols, axis=1)
    tmt_ref[0] = jnp.concatenate(tm_cols, axis=1)
    zin = jnp.concatenate(z_rows, axis=0)
    z_ref[...] = _dot(zin, projw_ref[...], (((1,), (0,))))


def kernel(x, conv_w, channel_embed, in_proj_w, in_proj_b, out_proj_w,
           out_proj_b, slots, proj_w):
    Bn = x.shape[0]
    patches = jax.lax.conv_general_dilated_patches(
        x, (9, 9), (2, 2), ((4, 4), (4, 4)),
        dimension_numbers=('NCHW', 'OIHW', 'NCHW'))
    pt = patches.reshape(Bn, 81, N)
    w2 = conv_w.reshape(C1, 81).T
    pe = _pos2d(H, W, D)

    grid = (Bn // BB,)
    outs = pl.pallas_call(
        _fused_kernel,
        grid=grid,
        in_specs=[
            pl.BlockSpec((BB, 81, N), lambda i: (i, 0, 0)),
            pl.BlockSpec((81, C1), lambda i: (0, 0)),
            pl.BlockSpec((C1, D), lambda i: (0, 0)),
            pl.BlockSpec((N, D), lambda i: (0, 0)),
            pl.BlockSpec((3 * D, D), lambda i: (0, 0)),
            pl.BlockSpec((1, 3 * D), lambda i: (0, 0)),
            pl.BlockSpec((D, D), lambda i: (0, 0)),
            pl.BlockSpec((1, D), lambda i: (0, 0)),
            pl.BlockSpec((MP, D), lambda i: (0, 0)),
            pl.BlockSpec((D, D), lambda i: (0, 0)),
        ],
        out_specs=[
            pl.BlockSpec((BB, D), lambda i: (i, 0)),
            pl.BlockSpec((BB, M, N), lambda i: (i, 0, 0)),
            pl.BlockSpec((1, M, BB), lambda i: (i, 0, 0)),
            pl.BlockSpec((BB, N, C1), lambda i: (i, 0, 0)),
            pl.BlockSpec((BB, N, KCH), lambda i: (i, 0, 0)),
            pl.BlockSpec((BB, M, D), lambda i: (i, 0, 0)),
            pl.BlockSpec((1, M, BB), lambda i: (i, 0, 0)),
        ],
        out_shape=[
            jax.ShapeDtypeStruct((Bn, D), jnp.float32),
            jax.ShapeDtypeStruct((Bn, M, N), jnp.float32),
            jax.ShapeDtypeStruct((Bn // BB, M, BB), jnp.float32),
            jax.ShapeDtypeStruct((Bn, N, C1), jnp.float32),
            jax.ShapeDtypeStruct((Bn, N, KCH), jnp.int32),
            jax.ShapeDtypeStruct((Bn, M, D), jnp.float32),
            jax.ShapeDtypeStruct((Bn // BB, M, BB), jnp.float32),
        ],
        compiler_params=pltpu.CompilerParams(
            dimension_semantics=("arbitrary",)),
    )(pt, w2, channel_embed, pe, in_proj_w, in_proj_b.reshape(1, 3 * D),
      out_proj_w, out_proj_b.reshape(1, D),
      jnp.concatenate([slots, jnp.zeros((MP - M, D), jnp.float32)], axis=0),
      proj_w)
    z, am, het, sw, ti, ss, tmt = outs
    sparse_weights = sw.reshape(Bn, H, W, C1)
    topi = ti.reshape(Bn, H, W, KCH)
    he = het.transpose(0, 2, 1).reshape(Bn, M)
    tm = tmt.transpose(0, 2, 1).reshape(Bn, M)
    return (z, am, he, sparse_weights, topi, ss, tm)


# speedup vs baseline: 1.1023x; 1.0658x over previous
import functools
import math
import jax
import jax.numpy as jnp
from jax.experimental import pallas as pl
from jax.experimental.pallas import tpu as pltpu

C1 = 150; D = 32; NH = 4; KCH = 10; M = 12; H = 14; W = 14; RATIO = 0.08
N = H * W
KGLOBAL = max(1, int(math.ceil(RATIO * (C1 * H * W))))
BB = 16
MP = 16


def _pos2d(h, w, dim):
    d2 = dim // 2
    div = jnp.exp(jnp.arange(0, d2, 2, dtype=jnp.float32) * -(math.log(10000.0) / d2))
    pos_y = jnp.arange(h, dtype=jnp.float32)[:, None]
    ang_y = pos_y * div[None, :]
    pe_y = jnp.zeros((h, d2), jnp.float32).at[:, 0::2].set(jnp.sin(ang_y)).at[:, 1::2].set(jnp.cos(ang_y))
    pos_x = jnp.arange(w, dtype=jnp.float32)[:, None]
    ang_x = pos_x * div[None, :]
    pe_x = jnp.zeros((w, d2), jnp.float32).at[:, 0::2].set(jnp.sin(ang_x)).at[:, 1::2].set(jnp.cos(ang_x))
    pe = jnp.concatenate([
        jnp.broadcast_to(pe_y[:, None, :], (h, w, d2)),
        jnp.broadcast_to(pe_x[None, :, :], (h, w, d2)),
    ], axis=-1)
    return pe.reshape(h * w, dim)


def _dot(a, b, dims):
    return jax.lax.dot_general(a, b, (dims, ((), ())),
                               preferred_element_type=jnp.float32)


def _fused_kernel(pt_ref, w2_ref, cemb_ref, pe_ref, inw_ref, inb_ref,
                  outw_ref, outb_ref, slots_ref, projw_ref,
                  z_ref, am_ref, het_ref, sw_ref, ti_ref, ss_ref, tmt_ref):
    bb = pt_ref.shape[0]
    w2 = w2_ref[...]

    a_list = []
    for j in range(bb):
        aj = _dot(pt_ref[j], w2, (((0,), (0,))))
        a_list.append(jnp.maximum(aj, 0.0))
    A = jnp.stack(a_list, axis=0)

    abits = jnp.maximum(jax.lax.bitcast_convert_type(A, jnp.int32), 0)
    gmaxb = jnp.max(abits, axis=(1, 2), keepdims=True)

    lo = jnp.zeros((bb, 1, 1), jnp.int32)
    hi = gmaxb + 1
    for _ in range(31):
        mid = lo + jax.lax.shift_right_logical(hi - lo, 1)
        cnt = jnp.sum(jnp.where(abits >= mid, 1.0, 0.0),
                      axis=(1, 2), keepdims=True)
        ge = cnt >= float(KGLOBAL)
        lo = jnp.where(ge, mid, lo)
        hi = jnp.where(ge, hi, mid)
    thresh = jax.lax.bitcast_convert_type(lo, jnp.float32)
    m0 = jnp.where(A >= thresh, A, 0.0)

    cif = jax.lax.broadcasted_iota(jnp.int32, (bb, N, C1), 2).astype(jnp.float32)
    m = m0
    idxs = []
    for _ in range(KCH):
        cur = jnp.max(m, axis=2, keepdims=True)
        idxf = jnp.min(jnp.where(m == cur, cif, jnp.float32(C1)),
                       axis=2, keepdims=True)
        idxs.append(idxf.astype(jnp.int32))
        m = jnp.where(cif == idxf, jnp.float32(-1.0), m)
    ti_ref[...] = jnp.concatenate(idxs, axis=2)

    gmaxf = jax.lax.bitcast_convert_type(gmaxb, jnp.float32)
    denom = jnp.where(gmaxf == 0.0, 1.0, gmaxf)
    sw = jnp.where(m < 0.0, m0, 0.0) / denom
    sw_ref[...] = sw

    cemb = cemb_ref[...]; pe = pe_ref[...]
    inw = inw_ref[...]; inb = inb_ref[...]
    outw = outw_ref[...]; outb = outb_ref[...]
    slots = slots_ref[...]
    dh = D // NH
    asc = 1.0 / math.sqrt(dh)
    ssc = 1.0 / math.sqrt(D)
    z_rows, he_cols, tm_cols = [], [], []
    for j in range(bb):
        tok = _dot(sw[j], cemb, (((1,), (0,)))) + pe
        qkv = _dot(tok, inw, (((1,), (1,)))) + inb
        heads = []
        for h in range(NH):
            q = qkv[:, dh * h:dh * h + dh]
            k = qkv[:, D + dh * h:D + dh * h + dh]
            v = qkv[:, 2 * D + dh * h:2 * D + dh * h + dh]
            sc = _dot(q, k, (((1,), (1,)))) * asc
            sc = sc - jnp.max(sc, axis=1, keepdims=True)
            e = jnp.exp(sc)
            p = e / jnp.sum(e, axis=1, keepdims=True)
            heads.append(_dot(p, v, (((1,), (0,)))))
        o = jnp.concatenate(heads, axis=1)
        ao = _dot(o, outw, (((1,), (1,)))) + outb
        st = _dot(slots, ao, (((1,), (1,)))) * ssc
        st = st - jnp.max(st, axis=1, keepdims=True)
        e = jnp.exp(st)
        am16 = e / jnp.sum(e, axis=1, keepdims=True)
        am = am16[:M]
        am_ref[j] = am
        he_cols.append(jnp.sum(am * am, axis=1, keepdims=True))
        ssj = _dot(am, ao, (((1,), (0,))))
        ss_ref[j] = ssj
        ni = jax.lax.broadcasted_iota(jnp.int32, (M, N), 1).astype(jnp.float32)
        mm = am
        acc = jnp.zeros((M, 1), jnp.float32)
        for _ in range(16):
            cur = jnp.max(mm, axis=1, keepdims=True)
            ii = jnp.min(jnp.where(mm == cur, ni, jnp.float32(N)),
                         axis=1, keepdims=True)
            acc = acc + cur
            mm = jnp.where(ni == ii, jnp.float32(-1.0), mm)
        tm_cols.append(acc)
        z_rows.append(jnp.mean(ssj, axis=0, keepdims=True))
    het_ref[0] = jnp.concatenate(he_cols, axis=1)
    tmt_ref[0] = jnp.concatenate(tm_cols, axis=1)
    zin = jnp.concatenate(z_rows, axis=0)
    z_ref[...] = _dot(zin, projw_ref[...], (((1,), (0,))))


def kernel(x, conv_w, channel_embed, in_proj_w, in_proj_b, out_proj_w,
           out_proj_b, slots, proj_w):
    Bn = x.shape[0]
    patches = jax.lax.conv_general_dilated_patches(
        x, (9, 9), (2, 2), ((4, 4), (4, 4)),
        dimension_numbers=('NCHW', 'OIHW', 'NCHW'))
    pt = patches.reshape(Bn, 81, N)
    w2 = conv_w.reshape(C1, 81).T
    pe = _pos2d(H, W, D)

    grid = (Bn // BB,)
    outs = pl.pallas_call(
        _fused_kernel,
        grid=grid,
        in_specs=[
            pl.BlockSpec((BB, 81, N), lambda i: (i, 0, 0)),
            pl.BlockSpec((81, C1), lambda i: (0, 0)),
            pl.BlockSpec((C1, D), lambda i: (0, 0)),
            pl.BlockSpec((N, D), lambda i: (0, 0)),
            pl.BlockSpec((3 * D, D), lambda i: (0, 0)),
            pl.BlockSpec((1, 3 * D), lambda i: (0, 0)),
            pl.BlockSpec((D, D), lambda i: (0, 0)),
            pl.BlockSpec((1, D), lambda i: (0, 0)),
            pl.BlockSpec((MP, D), lambda i: (0, 0)),
            pl.BlockSpec((D, D), lambda i: (0, 0)),
        ],
        out_specs=[
            pl.BlockSpec((BB, D), lambda i: (i, 0)),
            pl.BlockSpec((BB, M, N), lambda i: (i, 0, 0)),
            pl.BlockSpec((1, M, BB), lambda i: (i, 0, 0)),
            pl.BlockSpec((BB, N, C1), lambda i: (i, 0, 0)),
            pl.BlockSpec((BB, N, KCH), lambda i: (i, 0, 0)),
            pl.BlockSpec((BB, M, D), lambda i: (i, 0, 0)),
            pl.BlockSpec((1, M, BB), lambda i: (i, 0, 0)),
        ],
        out_shape=[
            jax.ShapeDtypeStruct((Bn, D), jnp.float32),
            jax.ShapeDtypeStruct((Bn, M, N), jnp.float32),
            jax.ShapeDtypeStruct((Bn // BB, M, BB), jnp.float32),
            jax.ShapeDtypeStruct((Bn, N, C1), jnp.float32),
            jax.ShapeDtypeStruct((Bn, N, KCH), jnp.int32),
            jax.ShapeDtypeStruct((Bn, M, D), jnp.float32),
            jax.ShapeDtypeStruct((Bn // BB, M, BB), jnp.float32),
        ],
        compiler_params=pltpu.CompilerParams(
            dimension_semantics=("arbitrary",)),
    )(pt, w2, channel_embed, pe, in_proj_w, in_proj_b.reshape(1, 3 * D),
      out_proj_w, out_proj_b.reshape(1, D),
      jnp.concatenate([slots, jnp.zeros((MP - M, D), jnp.float32)], axis=0),
      proj_w)
    z, am, het, sw, ti, ss, tmt = outs
    sparse_weights = sw.reshape(Bn, H, W, C1)
    topi = ti.reshape(Bn, H, W, KCH)
    he = het.transpose(0, 2, 1).reshape(Bn, M)
    tm = tmt.transpose(0, 2, 1).reshape(Bn, M)
    return (z, am, he, sparse_weights, topi, ss, tm)
